# baseline (speedup 1.0000x reference)
import jax
import jax.numpy as jnp
from jax.experimental import pallas as pl
from jax.experimental.pallas import tpu as pltpu

B, L, D, DO = 16, 2048, 1024, 128
TL = 256
NB = L // TL
GMAX = B * NB
NBUF = 4


def _body(bag_ref, jj_ref, cnt_ref, n_ref,
          x_hbm, W1_ref, b1_ref, W2_ref, b2_ref,
          out_ref, buf, acc_ref, sems):
    total = cnt_ref[0]

    def dma(g):
        slot = jax.lax.rem(g, NBUF)
        bg = bag_ref[g]
        jg = jj_ref[g]
        return pltpu.make_async_copy(
            x_hbm.at[bg, pl.ds(jg * TL, TL), :],
            buf.at[slot],
            sems.at[slot],
        )

    def prime(k, carry):
        @pl.when(k < total)
        def _():
            dma(k).start()
        return carry

    jax.lax.fori_loop(0, NBUF - 1, prime, 0)

    def step(g, carry):
        @pl.when(g + NBUF - 1 < total)
        def _():
            dma(g + NBUF - 1).start()

        dma(g).wait()
        slot = jax.lax.rem(g, NBUF)
        bg = bag_ref[g]
        jg = jj_ref[g]
        nb = n_ref[bg]

        xb = buf[slot]
        y = jnp.dot(xb, W1_ref[...], preferred_element_type=jnp.float32)
        y = jnp.maximum(y + b1_ref[...], 0.0)
        rows = jg * TL + jax.lax.broadcasted_iota(jnp.int32, (TL, 1), 0)
        y = jnp.where(rows < nb, y, 0.0)
        s = jnp.sum(y, axis=0, keepdims=True)
        prev = jnp.where(jg == 0, jnp.zeros_like(s), acc_ref[...])
        tot = prev + s
        acc_ref[...] = tot

        @pl.when((jg + 1) * TL >= nb)
        def _():
            pooled = tot / nb.astype(jnp.float32)
            out_ref[pl.ds(bg, 1), :] = (
                jnp.dot(pooled, W2_ref[...], preferred_element_type=jnp.float32)
                + b2_ref[...]
            )

        return carry

    jax.lax.fori_loop(0, total, step, 0)


def kernel(x, n_instances, W1, b1, W2, b2):
    n = n_instances.astype(jnp.int32)
    b1r = b1.reshape(1, D)
    b2r = b2.reshape(1, DO)

    nb = (n + TL - 1) // TL
    ends = jnp.cumsum(nb)
    total = ends[B - 1].reshape(1).astype(jnp.int32)
    g = jnp.arange(GMAX, dtype=jnp.int32)
    bag_of_g = jnp.minimum(
        jnp.searchsorted(ends, g, side="right").astype(jnp.int32), B - 1)
    j_of_g = g - (ends - nb)[bag_of_g]

    grid_spec = pltpu.PrefetchScalarGridSpec(
        num_scalar_prefetch=4,
        grid=(1,),
        in_specs=[
            pl.BlockSpec(memory_space=pl.ANY),
            pl.BlockSpec((D, D), lambda i, *_: (0, 0)),
            pl.BlockSpec((1, D), lambda i, *_: (0, 0)),
            pl.BlockSpec((D, DO), lambda i, *_: (0, 0)),
            pl.BlockSpec((1, DO), lambda i, *_: (0, 0)),
        ],
        out_specs=pl.BlockSpec((B, DO), lambda i, *_: (0, 0)),
        scratch_shapes=[
            pltpu.VMEM((NBUF, TL, D), jnp.float32),
            pltpu.VMEM((1, D), jnp.float32),
            pltpu.SemaphoreType.DMA((NBUF,)),
        ],
    )

    return pl.pallas_call(
        _body,
        grid_spec=grid_spec,
        out_shape=jax.ShapeDtypeStruct((B, DO), jnp.float32),
        compiler_params=pltpu.CompilerParams(
            dimension_semantics=("arbitrary",),
        ),
    )(bag_of_g, j_of_g, total, n, x, W1, b1r, W2, b2r)


# speedup vs baseline: 1.0084x; 1.0084x over previous
import jax
import jax.numpy as jnp
from jax.experimental import pallas as pl
from jax.experimental.pallas import tpu as pltpu

B, L, D, DO = 16, 2048, 1024, 128
TL = 256
NB = L // TL
GMAX = B * NB
NBUF = 4


def _body(bag_ref, jj_ref, cnt_ref, n_ref,
          x_hbm, W1_ref, b1_ref, W2_ref, b2_ref,
          out_ref, buf, acc_ref, sems):
    total = cnt_ref[0]

    def dma(g, slot):
        bg = bag_ref[g]
        jg = jj_ref[g]
        return pltpu.make_async_copy(
            x_hbm.at[bg, pl.ds(jg * TL, TL), :],
            buf.at[slot],
            sems.at[slot],
        )

    for k in range(NBUF - 1):
        @pl.when(k < total)
        def _(k=k):
            dma(k, k).start()

    def compute(g, k):
        dma(g, k).wait()
        bg = bag_ref[g]
        jg = jj_ref[g]
        nb = n_ref[bg]

        xb = buf[k]
        y = jnp.dot(xb, W1_ref[...], preferred_element_type=jnp.float32)
        y = jnp.maximum(y + b1_ref[...], 0.0)
        rows = jg * TL + jax.lax.broadcasted_iota(jnp.int32, (TL, 1), 0)
        y = jnp.where(rows < nb, y, 0.0)
        s = jnp.sum(y, axis=0, keepdims=True)
        prev = jnp.where(jg == 0, jnp.zeros_like(s), acc_ref[...])
        tot = prev + s
        acc_ref[...] = tot

        @pl.when((jg + 1) * TL >= nb)
        def _():
            pooled = tot / nb.astype(jnp.float32)
            out_ref[pl.ds(bg, 1), :] = (
                jnp.dot(pooled, W2_ref[...], preferred_element_type=jnp.float32)
                + b2_ref[...]
            )

    def outer(i, carry):
        g0 = i * NBUF
        for k in range(NBUF):
            g = g0 + k

            @pl.when(g < total)
            def _(g=g, k=k):
                @pl.when(g + NBUF - 1 < total)
                def _():
                    dma(g + NBUF - 1, (k + NBUF - 1) % NBUF).start()

                compute(g, k)

        return carry

    jax.lax.fori_loop(0, (total + NBUF - 1) // NBUF, outer, 0)


def kernel(x, n_instances, W1, b1, W2, b2):
    n = n_instances.astype(jnp.int32)
    b1r = b1.reshape(1, D)
    b2r = b2.reshape(1, DO)

    nb = (n + TL - 1) // TL
    ends = jnp.cumsum(nb)
    total = ends[B - 1].reshape(1).astype(jnp.int32)
    g = jnp.arange(GMAX, dtype=jnp.int32)
    bag_of_g = jnp.minimum(
        jnp.searchsorted(ends, g, side="right").astype(jnp.int32), B - 1)
    j_of_g = g - (ends - nb)[bag_of_g]

    grid_spec = pltpu.PrefetchScalarGridSpec(
        num_scalar_prefetch=4,
        grid=(1,),
        in_specs=[
            pl.BlockSpec(memory_space=pl.ANY),
            pl.BlockSpec((D, D), lambda i, *_: (0, 0)),
            pl.BlockSpec((1, D), lambda i, *_: (0, 0)),
            pl.BlockSpec((D, DO), lambda i, *_: (0, 0)),
            pl.BlockSpec((1, DO), lambda i, *_: (0, 0)),
        ],
        out_specs=pl.BlockSpec((B, DO), lambda i, *_: (0, 0)),
        scratch_shapes=[
            pltpu.VMEM((NBUF, TL, D), jnp.float32),
            pltpu.VMEM((1, D), jnp.float32),
            pltpu.SemaphoreType.DMA((NBUF,)),
        ],
    )

    return pl.pallas_call(
        _body,
        grid_spec=grid_spec,
        out_shape=jax.ShapeDtypeStruct((B, DO), jnp.float32),
        compiler_params=pltpu.CompilerParams(
            dimension_semantics=("arbitrary",),
        ),
    )(bag_of_g, j_of_g, total, n, x, W1, b1r, W2, b2r)
